# initial kernel scaffold (unmeasured)
import jax
import jax.numpy as jnp
from jax import lax
from jax.experimental import pallas as pl
from jax.experimental.pallas import tpu as pltpu

N_DEV = 4


def kernel(x, Wg, Wu, Wd):
    x = x.astype(jnp.bfloat16)
    Wg = Wg.astype(jnp.bfloat16)
    Wu = Wu.astype(jnp.bfloat16)
    Wd = Wd.astype(jnp.bfloat16)

    m, d_model = x.shape
    chunk = m // N_DEV
    n_hops = 2 * (N_DEV - 1)

    def body(x_ref, wg_ref, wu_ref, wd_ref, out_ref,
             send_buf, recv_buf, send_sems, recv_sems):
        my = lax.axis_index("i")
        left = lax.rem(my + N_DEV - 1, N_DEV)
        right = lax.rem(my + 1, N_DEV)

        barrier_sem = pltpu.get_barrier_semaphore()
        for nbr in (left, right):
            pl.semaphore_signal(
                barrier_sem, inc=1,
                device_id=(nbr,), device_id_type=pl.DeviceIdType.MESH,
            )
        pl.semaphore_wait(barrier_sem, 2)

        for c in range(N_DEV):
            rows = pl.ds(c * chunk, chunk)
            xc = x_ref[rows, :]
            gate = jnp.dot(xc, wg_ref[:, :], preferred_element_type=jnp.float32)
            up = jnp.dot(xc, wu_ref[:, :], preferred_element_type=jnp.float32)
            h = (gate * (up * jax.nn.sigmoid(up))).astype(jnp.bfloat16)
            out_ref[rows, :] = jnp.dot(
                h, wd_ref[:, :], preferred_element_type=jnp.float32
            )

        def hop(slot, c_send, c_recv, accumulate):
            send_buf[slot, :, :] = out_ref[
                pl.ds(c_send * chunk, chunk), :
            ].astype(jnp.bfloat16)
            rdma = pltpu.make_async_remote_copy(
                src_ref=send_buf.at[slot],
                dst_ref=recv_buf.at[slot],
                send_sem=send_sems.at[slot],
                recv_sem=recv_sems.at[slot],
                device_id=(right,),
                device_id_type=pl.DeviceIdType.MESH,
            )
            rdma.start()
            rdma.wait()
            got = recv_buf[slot, :, :].astype(jnp.float32)
            dst = pl.ds(c_recv * chunk, chunk)
            if accumulate:
                out_ref[dst, :] += got
            else:
                out_ref[dst, :] = got

        for s in range(N_DEV - 1):
            hop(s,
                lax.rem(my - s + N_DEV, N_DEV),
                lax.rem(my - s - 1 + N_DEV, N_DEV),
                accumulate=True)

        for s in range(N_DEV - 1):
            hop(N_DEV - 1 + s,
                lax.rem(my + 1 - s + N_DEV, N_DEV),
                lax.rem(my - s + N_DEV, N_DEV),
                accumulate=False)

    return pl.pallas_call(
        body,
        out_shape=jax.ShapeDtypeStruct((m, d_model), jnp.float32),
        in_specs=[pl.BlockSpec(memory_space=pltpu.VMEM)] * 4,
        out_specs=pl.BlockSpec(memory_space=pltpu.VMEM),
        scratch_shapes=[
            pltpu.VMEM((n_hops, chunk, d_model), jnp.bfloat16),
            pltpu.VMEM((n_hops, chunk, d_model), jnp.bfloat16),
            pltpu.SemaphoreType.DMA((n_hops,)),
            pltpu.SemaphoreType.DMA((n_hops,)),
        ],
        compiler_params=pltpu.CompilerParams(collective_id=0),
    )(x, Wg, Wu, Wd)


# baseline (device time: 191871 ns/iter reference)
import jax
import jax.numpy as jnp
from jax import lax
from jax.experimental import pallas as pl
from jax.experimental.pallas import tpu as pltpu

N_DEV = 4


def kernel(x, Wg, Wu, Wd):
    x = x.astype(jnp.bfloat16)
    Wg = Wg.astype(jnp.bfloat16)
    Wu = Wu.astype(jnp.bfloat16)
    Wd = Wd.astype(jnp.bfloat16)

    m, d_model = x.shape
    chunk = m // N_DEV
    n_hops = 2 * (N_DEV - 1)

    def body(x_ref, wg_ref, wu_ref, wd_ref, out_ref,
             send_buf, recv_buf, send_sems, recv_sems):
        my = lax.axis_index("i")
        left = lax.rem(my + N_DEV - 1, N_DEV)
        right = lax.rem(my + 1, N_DEV)

        barrier_sem = pltpu.get_barrier_semaphore()
        for nbr in (left, right):
            pl.semaphore_signal(
                barrier_sem, inc=1,
                device_id=(nbr,), device_id_type=pl.DeviceIdType.MESH,
            )
        pl.semaphore_wait(barrier_sem, 2)

        for c in range(N_DEV):
            rows = pl.ds(c * chunk, chunk)
            xc = x_ref[rows, :]
            gate = jnp.dot(xc, wg_ref[:, :], preferred_element_type=jnp.float32)
            up = jnp.dot(xc, wu_ref[:, :], preferred_element_type=jnp.float32)
            h = (gate * (up * jax.nn.sigmoid(up))).astype(jnp.bfloat16)
            out_ref[rows, :] = jnp.dot(
                h, wd_ref[:, :], preferred_element_type=jnp.float32
            )

        def hop(slot, c_send, c_recv, accumulate):
            send_buf[slot, :, :] = out_ref[
                pl.ds(c_send * chunk, chunk), :
            ].astype(jnp.bfloat16)
            rdma = pltpu.make_async_remote_copy(
                src_ref=send_buf.at[slot],
                dst_ref=recv_buf.at[slot],
                send_sem=send_sems.at[slot],
                recv_sem=recv_sems.at[slot],
                device_id=(right,),
                device_id_type=pl.DeviceIdType.MESH,
            )
            rdma.start()
            rdma.wait()
            got = recv_buf[slot, :, :].astype(jnp.float32)
            dst = pl.ds(c_recv * chunk, chunk)
            if accumulate:
                out_ref[dst, :] += got
            else:
                out_ref[dst, :] = got

        for s in range(N_DEV - 1):
            hop(s,
                lax.rem(my - s + N_DEV, N_DEV),
                lax.rem(my - s - 1 + N_DEV, N_DEV),
                accumulate=True)

        for s in range(N_DEV - 1):
            hop(N_DEV - 1 + s,
                lax.rem(my + 1 - s + N_DEV, N_DEV),
                lax.rem(my - s + N_DEV, N_DEV),
                accumulate=False)

    return pl.pallas_call(
        body,
        out_shape=jax.ShapeDtypeStruct((m, d_model), jnp.float32),
        in_specs=[pl.BlockSpec(memory_space=pltpu.VMEM)] * 4,
        out_specs=pl.BlockSpec(memory_space=pltpu.VMEM),
        scratch_shapes=[
            pltpu.VMEM((n_hops, chunk, d_model), jnp.bfloat16),
            pltpu.VMEM((n_hops, chunk, d_model), jnp.bfloat16),
            pltpu.SemaphoreType.DMA((n_hops,)),
            pltpu.SemaphoreType.DMA((n_hops,)),
        ],
        compiler_params=pltpu.CompilerParams(
            collective_id=0,
            vmem_limit_bytes=100 * 1024 * 1024,
        ),
    )(x, Wg, Wu, Wd)


# device time: 156760 ns/iter; 1.2240x vs baseline; 1.2240x over previous
import jax
import jax.numpy as jnp
from jax import lax
from jax.experimental import pallas as pl
from jax.experimental.pallas import tpu as pltpu

N_DEV = 4


def kernel(x, Wg, Wu, Wd):
    x = x.astype(jnp.bfloat16)
    Wg = Wg.astype(jnp.bfloat16)
    Wu = Wu.astype(jnp.bfloat16)
    Wd = Wd.astype(jnp.bfloat16)

    m, d_model = x.shape
    chunk = m // N_DEV
    n_hops = 2 * (N_DEV - 1)

    def body(x_ref, wg_ref, wu_ref, wd_ref, out_ref,
             send_buf, recv_buf, send_sems, recv_sems):
        my = lax.axis_index("i")
        left = lax.rem(my + N_DEV - 1, N_DEV)
        right = lax.rem(my + 1, N_DEV)

        barrier_sem = pltpu.get_barrier_semaphore()
        for nbr in (left, right):
            pl.semaphore_signal(
                barrier_sem, inc=1,
                device_id=(nbr,), device_id_type=pl.DeviceIdType.MESH,
            )
        pl.semaphore_wait(barrier_sem, 2)

        def compute_chunk(c):
            rows = pl.ds(c * chunk, chunk)
            xc = x_ref[rows, :]
            gate = jnp.dot(xc, wg_ref[:, :], preferred_element_type=jnp.float32)
            up = jnp.dot(xc, wu_ref[:, :], preferred_element_type=jnp.float32)
            h = (gate * (up * jax.nn.sigmoid(up))).astype(jnp.bfloat16)
            out_ref[rows, :] = jnp.dot(
                h, wd_ref[:, :], preferred_element_type=jnp.float32
            )

        def make_hop(slot, c_send):
            send_buf[slot, :, :] = out_ref[
                pl.ds(c_send * chunk, chunk), :
            ].astype(jnp.bfloat16)
            rdma = pltpu.make_async_remote_copy(
                src_ref=send_buf.at[slot],
                dst_ref=recv_buf.at[slot],
                send_sem=send_sems.at[slot],
                recv_sem=recv_sems.at[slot],
                device_id=(right,),
                device_id_type=pl.DeviceIdType.MESH,
            )
            rdma.start()
            return rdma

        compute_chunk(lax.rem(my, N_DEV))
        for s in range(N_DEV - 1):
            c_send = lax.rem(my - s + N_DEV, N_DEV)
            c_recv = lax.rem(my - s - 1 + N_DEV, N_DEV)
            rdma = make_hop(s, c_send)
            compute_chunk(c_recv)
            rdma.wait()
            dst = pl.ds(c_recv * chunk, chunk)
            out_ref[dst, :] += recv_buf[s, :, :].astype(jnp.float32)

        for s in range(N_DEV - 1):
            c_send = lax.rem(my + 1 - s + N_DEV, N_DEV)
            c_recv = lax.rem(my - s + N_DEV, N_DEV)
            rdma = make_hop(N_DEV - 1 + s, c_send)
            rdma.wait()
            dst = pl.ds(c_recv * chunk, chunk)
            out_ref[dst, :] = recv_buf[N_DEV - 1 + s, :, :].astype(jnp.float32)

    return pl.pallas_call(
        body,
        out_shape=jax.ShapeDtypeStruct((m, d_model), jnp.float32),
        in_specs=[pl.BlockSpec(memory_space=pltpu.VMEM)] * 4,
        out_specs=pl.BlockSpec(memory_space=pltpu.VMEM),
        scratch_shapes=[
            pltpu.VMEM((n_hops, chunk, d_model), jnp.bfloat16),
            pltpu.VMEM((n_hops, chunk, d_model), jnp.bfloat16),
            pltpu.SemaphoreType.DMA((n_hops,)),
            pltpu.SemaphoreType.DMA((n_hops,)),
        ],
        compiler_params=pltpu.CompilerParams(
            collective_id=0,
            vmem_limit_bytes=100 * 1024 * 1024,
        ),
    )(x, Wg, Wu, Wd)


# device time: 138082 ns/iter; 1.3895x vs baseline; 1.1353x over previous
import jax
import jax.numpy as jnp
from jax import lax
from jax.experimental import pallas as pl
from jax.experimental.pallas import tpu as pltpu

N_DEV = 4


def kernel(x, Wg, Wu, Wd):
    x = x.astype(jnp.bfloat16)
    Wg = Wg.astype(jnp.bfloat16)
    Wu = Wu.astype(jnp.bfloat16)
    Wd = Wd.astype(jnp.bfloat16)

    m, d_model = x.shape
    chunk = m // N_DEV
    half = d_model // 2
    n_rs = N_DEV - 1
    n_ag = N_DEV - 1

    def body(x_ref, wg_ref, wu_ref, wd_ref, out_ref,
             send_buf, recv_buf, send_sems, recv_sems,
             ag_own_r, ag_own_l, ag_recv_r, ag_recv_l,
             ag_ssem_r, ag_rsem_r, ag_ssem_l, ag_rsem_l):
        my = lax.axis_index("i")
        left = lax.rem(my + N_DEV - 1, N_DEV)
        right = lax.rem(my + 1, N_DEV)

        barrier_sem = pltpu.get_barrier_semaphore()
        for nbr in (left, right):
            pl.semaphore_signal(
                barrier_sem, inc=1,
                device_id=(nbr,), device_id_type=pl.DeviceIdType.MESH,
            )
        pl.semaphore_wait(barrier_sem, 2)

        def compute_chunk(c):
            rows = pl.ds(c * chunk, chunk)
            xc = x_ref[rows, :]
            gate = jnp.dot(xc, wg_ref[:, :], preferred_element_type=jnp.float32)
            up = jnp.dot(xc, wu_ref[:, :], preferred_element_type=jnp.float32)
            h = (gate * (up * jax.nn.sigmoid(up))).astype(jnp.bfloat16)
            out_ref[rows, :] = jnp.dot(
                h, wd_ref[:, :], preferred_element_type=jnp.float32
            )

        def make_hop(slot, c_send):
            send_buf[slot, :, :] = out_ref[
                pl.ds(c_send * chunk, chunk), :
            ].astype(jnp.bfloat16)
            rdma = pltpu.make_async_remote_copy(
                src_ref=send_buf.at[slot],
                dst_ref=recv_buf.at[slot],
                send_sem=send_sems.at[slot],
                recv_sem=recv_sems.at[slot],
                device_id=(right,),
                device_id_type=pl.DeviceIdType.MESH,
            )
            rdma.start()
            return rdma

        compute_chunk(lax.rem(my, N_DEV))
        for s in range(N_DEV - 1):
            c_send = lax.rem(my - s + N_DEV, N_DEV)
            c_recv = lax.rem(my - s - 1 + N_DEV, N_DEV)
            rdma = make_hop(s, c_send)
            compute_chunk(c_recv)
            rdma.wait()
            dst = pl.ds(c_recv * chunk, chunk)
            out_ref[dst, :] += recv_buf[s, :, :].astype(jnp.float32)

        red = pl.ds(lax.rem(my + 1, N_DEV) * chunk, chunk)
        ag_own_r[:, :] = out_ref[red, 0:half].astype(jnp.bfloat16)
        ag_own_l[:, :] = out_ref[red, half:].astype(jnp.bfloat16)

        for s in range(n_ag):
            src_r = ag_own_r if s == 0 else ag_recv_r.at[s - 1]
            src_l = ag_own_l if s == 0 else ag_recv_l.at[s - 1]
            rdma_r = pltpu.make_async_remote_copy(
                src_ref=src_r,
                dst_ref=ag_recv_r.at[s],
                send_sem=ag_ssem_r.at[s],
                recv_sem=ag_rsem_r.at[s],
                device_id=(right,),
                device_id_type=pl.DeviceIdType.MESH,
            )
            rdma_l = pltpu.make_async_remote_copy(
                src_ref=src_l,
                dst_ref=ag_recv_l.at[s],
                send_sem=ag_ssem_l.at[s],
                recv_sem=ag_rsem_l.at[s],
                device_id=(left,),
                device_id_type=pl.DeviceIdType.MESH,
            )
            rdma_r.start()
            rdma_l.start()
            rdma_r.wait()
            rdma_l.wait()
            c_recv_r = lax.rem(my - s + N_DEV, N_DEV)
            c_recv_l = lax.rem(my + 2 + s, N_DEV)
            out_ref[pl.ds(c_recv_r * chunk, chunk), 0:half] = (
                ag_recv_r[s, :, :].astype(jnp.float32))
            out_ref[pl.ds(c_recv_l * chunk, chunk), half:] = (
                ag_recv_l[s, :, :].astype(jnp.float32))

    return pl.pallas_call(
        body,
        out_shape=jax.ShapeDtypeStruct((m, d_model), jnp.float32),
        in_specs=[pl.BlockSpec(memory_space=pltpu.VMEM)] * 4,
        out_specs=pl.BlockSpec(memory_space=pltpu.VMEM),
        scratch_shapes=[
            pltpu.VMEM((n_rs, chunk, d_model), jnp.bfloat16),
            pltpu.VMEM((n_rs, chunk, d_model), jnp.bfloat16),
            pltpu.SemaphoreType.DMA((n_rs,)),
            pltpu.SemaphoreType.DMA((n_rs,)),
            pltpu.VMEM((chunk, half), jnp.bfloat16),
            pltpu.VMEM((chunk, half), jnp.bfloat16),
            pltpu.VMEM((n_ag, chunk, half), jnp.bfloat16),
            pltpu.VMEM((n_ag, chunk, half), jnp.bfloat16),
            pltpu.SemaphoreType.DMA((n_ag,)),
            pltpu.SemaphoreType.DMA((n_ag,)),
            pltpu.SemaphoreType.DMA((n_ag,)),
            pltpu.SemaphoreType.DMA((n_ag,)),
        ],
        compiler_params=pltpu.CompilerParams(
            collective_id=0,
            vmem_limit_bytes=100 * 1024 * 1024,
        ),
    )(x, Wg, Wu, Wd)


# device time: 132375 ns/iter; 1.4495x vs baseline; 1.0431x over previous
import jax
import jax.numpy as jnp
from jax import lax
from jax.experimental import pallas as pl
from jax.experimental.pallas import tpu as pltpu

N_DEV = 4


def kernel(x, Wg, Wu, Wd):
    x = x.astype(jnp.bfloat16)
    Wg = Wg.astype(jnp.bfloat16)
    Wu = Wu.astype(jnp.bfloat16)
    Wd = Wd.astype(jnp.bfloat16)

    m, d_model = x.shape
    chunk = m // N_DEV
    half = d_model // 2
    n_rs = N_DEV - 1
    n_ag = N_DEV - 1

    def body(x_ref, wg_ref, wu_ref, wd_ref, out_ref,
             recv_buf, send_sems, recv_sems,
             ag_ssem_r, ag_rsem_r, ag_ssem_l, ag_rsem_l):
        my = lax.axis_index("i")
        left = lax.rem(my + N_DEV - 1, N_DEV)
        right = lax.rem(my + 1, N_DEV)

        barrier_sem = pltpu.get_barrier_semaphore()
        for nbr in (left, right):
            pl.semaphore_signal(
                barrier_sem, inc=1,
                device_id=(nbr,), device_id_type=pl.DeviceIdType.MESH,
            )
        pl.semaphore_wait(barrier_sem, 2)

        def compute_chunk(c):
            rows = pl.ds(c * chunk, chunk)
            xc = x_ref[rows, :]
            gate = jnp.dot(xc, wg_ref[:, :], preferred_element_type=jnp.float32)
            up = jnp.dot(xc, wu_ref[:, :], preferred_element_type=jnp.float32)
            h = (gate * (up * jax.nn.sigmoid(up))).astype(jnp.bfloat16)
            out_ref[rows, :] = jnp.dot(
                h, wd_ref[:, :], preferred_element_type=jnp.float32
            ).astype(jnp.bfloat16)

        compute_chunk(lax.rem(my, N_DEV))
        for s in range(n_rs):
            c_send = lax.rem(my - s + N_DEV, N_DEV)
            c_recv = lax.rem(my - s - 1 + N_DEV, N_DEV)
            rdma = pltpu.make_async_remote_copy(
                src_ref=out_ref.at[pl.ds(c_send * chunk, chunk), :],
                dst_ref=recv_buf.at[s],
                send_sem=send_sems.at[s],
                recv_sem=recv_sems.at[s],
                device_id=(right,),
                device_id_type=pl.DeviceIdType.MESH,
            )
            rdma.start()
            compute_chunk(c_recv)
            rdma.wait()
            dst = pl.ds(c_recv * chunk, chunk)
            out_ref[dst, :] = out_ref[dst, :] + recv_buf[s, :, :]

        for s in range(n_ag):
            c_r = lax.rem(my + 1 - s + N_DEV, N_DEV)
            c_l = lax.rem(my + 1 + s, N_DEV)
            rows_r = pl.ds(c_r * chunk, chunk)
            rows_l = pl.ds(c_l * chunk, chunk)
            rdma_r = pltpu.make_async_remote_copy(
                src_ref=out_ref.at[rows_r, 0:half],
                dst_ref=out_ref.at[rows_r, 0:half],
                send_sem=ag_ssem_r.at[s],
                recv_sem=ag_rsem_r.at[s],
                device_id=(right,),
                device_id_type=pl.DeviceIdType.MESH,
            )
            rdma_l = pltpu.make_async_remote_copy(
                src_ref=out_ref.at[rows_l, half:],
                dst_ref=out_ref.at[rows_l, half:],
                send_sem=ag_ssem_l.at[s],
                recv_sem=ag_rsem_l.at[s],
                device_id=(left,),
                device_id_type=pl.DeviceIdType.MESH,
            )
            rdma_r.start()
            rdma_l.start()
            rdma_r.wait()
            rdma_l.wait()

    return pl.pallas_call(
        body,
        out_shape=jax.ShapeDtypeStruct((m, d_model), jnp.bfloat16),
        in_specs=[pl.BlockSpec(memory_space=pltpu.VMEM)] * 4,
        out_specs=pl.BlockSpec(memory_space=pltpu.VMEM),
        scratch_shapes=[
            pltpu.VMEM((n_rs, chunk, d_model), jnp.bfloat16),
            pltpu.SemaphoreType.DMA((n_rs,)),
            pltpu.SemaphoreType.DMA((n_rs,)),
            pltpu.SemaphoreType.DMA((n_ag,)),
            pltpu.SemaphoreType.DMA((n_ag,)),
            pltpu.SemaphoreType.DMA((n_ag,)),
            pltpu.SemaphoreType.DMA((n_ag,)),
        ],
        compiler_params=pltpu.CompilerParams(
            collective_id=0,
            vmem_limit_bytes=62 * 1024 * 1024,
        ),
    )(x, Wg, Wu, Wd)


# device time: 112107 ns/iter; 1.7115x vs baseline; 1.1808x over previous
import jax
import jax.numpy as jnp
from jax import lax
from jax.experimental import pallas as pl
from jax.experimental.pallas import tpu as pltpu

N_DEV = 4


def kernel(x, Wg, Wu, Wd):
    m, d_model = x.shape
    hid = Wg.shape[1]
    wp = hid // 4
    chunk = m // N_DEV
    dp = chunk
    half = d_model // 2
    n_rs = N_DEV - 1
    n_ag = N_DEV - 1

    def body(x_hbm, wg_hbm, wu_hbm, wd_hbm, out_ref,
             wg_bf, wu_bf, wd_bf, stage_w, stage_d, stage_x,
             wsem, dsem, xsem,
             recv_buf, send_sems, recv_sems,
             ag_ssem_r, ag_rsem_r, ag_ssem_l, ag_rsem_l):
        my = lax.axis_index("i")
        left = lax.rem(my + N_DEV - 1, N_DEV)
        right = lax.rem(my + 1, N_DEV)

        barrier_sem = pltpu.get_barrier_semaphore()
        for nbr in (left, right):
            pl.semaphore_signal(
                barrier_sem, inc=1,
                device_id=(nbr,), device_id_type=pl.DeviceIdType.MESH,
            )
        pl.semaphore_wait(barrier_sem, 2)

        def w_dma(idx):
            ref = wg_hbm if idx < 4 else wu_hbm
            j = idx % 4
            return pltpu.make_async_copy(
                ref.at[:, j * wp:(j + 1) * wp], stage_w.at[idx % 2],
                wsem.at[idx % 2])

        def w_convert(idx):
            dst = wg_bf if idx < 4 else wu_bf
            j = idx % 4
            dst[:, j * wp:(j + 1) * wp] = stage_w[idx % 2].astype(jnp.bfloat16)

        def d_dma(idx):
            return pltpu.make_async_copy(
                wd_hbm.at[idx * dp:(idx + 1) * dp, :], stage_d.at[idx % 2],
                dsem.at[idx % 2])

        def d_convert(idx):
            wd_bf[idx * dp:(idx + 1) * dp, :] = (
                stage_d[idx % 2].astype(jnp.bfloat16))

        def x_dma(c, k):
            return pltpu.make_async_copy(
                x_hbm.at[pl.ds(c * chunk, chunk), :], stage_x.at[k % 2],
                xsem.at[k % 2])

        w_dma(0).start()
        w_dma(1).start()
        d_dma(0).start()
        d_dma(1).start()
        for idx in range(8):
            w_dma(idx).wait()
            w_convert(idx)
            if idx + 2 < 8:
                w_dma(idx + 2).start()

        chunk_order = [lax.rem(my - k + N_DEV, N_DEV) for k in range(N_DEV)]
        x_dma(chunk_order[0], 0).start()
        x_dma(chunk_order[1], 1).start()

        def compute_chunk(k):
            c = chunk_order[k]
            x_dma(c, k).wait()
            xc = stage_x[k % 2].astype(jnp.bfloat16)
            if k + 2 < N_DEV:
                x_dma(chunk_order[k + 2], k + 2).start()
            gate = jnp.dot(xc, wg_bf[:, :], preferred_element_type=jnp.float32)
            up = jnp.dot(xc, wu_bf[:, :], preferred_element_type=jnp.float32)
            h = (gate * (up * jax.nn.sigmoid(up))).astype(jnp.bfloat16)
            if k == 0:
                for idx in range(8):
                    d_dma(idx).wait()
                    d_convert(idx)
                    if idx + 2 < 8:
                        d_dma(idx + 2).start()
            out_ref[pl.ds(c * chunk, chunk), :] = jnp.dot(
                h, wd_bf[:, :], preferred_element_type=jnp.float32
            ).astype(jnp.bfloat16)

        compute_chunk(0)
        for s in range(n_rs):
            c_send = chunk_order[s]
            c_recv = chunk_order[s + 1]
            rdma = pltpu.make_async_remote_copy(
                src_ref=out_ref.at[pl.ds(c_send * chunk, chunk), :],
                dst_ref=recv_buf.at[s],
                send_sem=send_sems.at[s],
                recv_sem=recv_sems.at[s],
                device_id=(right,),
                device_id_type=pl.DeviceIdType.MESH,
            )
            rdma.start()
            compute_chunk(s + 1)
            rdma.wait()
            dst = pl.ds(c_recv * chunk, chunk)
            out_ref[dst, :] = out_ref[dst, :] + recv_buf[s, :, :]

        for s in range(n_ag):
            c_r = lax.rem(my + 1 - s + N_DEV, N_DEV)
            c_l = lax.rem(my + 1 + s, N_DEV)
            rows_r = pl.ds(c_r * chunk, chunk)
            rows_l = pl.ds(c_l * chunk, chunk)
            rdma_r = pltpu.make_async_remote_copy(
                src_ref=out_ref.at[rows_r, 0:half],
                dst_ref=out_ref.at[rows_r, 0:half],
                send_sem=ag_ssem_r.at[s],
                recv_sem=ag_rsem_r.at[s],
                device_id=(right,),
                device_id_type=pl.DeviceIdType.MESH,
            )
            rdma_l = pltpu.make_async_remote_copy(
                src_ref=out_ref.at[rows_l, half:],
                dst_ref=out_ref.at[rows_l, half:],
                send_sem=ag_ssem_l.at[s],
                recv_sem=ag_rsem_l.at[s],
                device_id=(left,),
                device_id_type=pl.DeviceIdType.MESH,
            )
            rdma_r.start()
            rdma_l.start()
            rdma_r.wait()
            rdma_l.wait()

    return pl.pallas_call(
        body,
        out_shape=jax.ShapeDtypeStruct((m, d_model), jnp.bfloat16),
        in_specs=[pl.BlockSpec(memory_space=pl.ANY)] * 4,
        out_specs=pl.BlockSpec(memory_space=pltpu.VMEM),
        scratch_shapes=[
            pltpu.VMEM((m, hid), jnp.bfloat16),
            pltpu.VMEM((m, hid), jnp.bfloat16),
            pltpu.VMEM((hid, d_model), jnp.bfloat16),
            pltpu.VMEM((2, m, wp), jnp.float32),
            pltpu.VMEM((2, dp, d_model), jnp.float32),
            pltpu.VMEM((2, chunk, d_model), jnp.float32),
            pltpu.SemaphoreType.DMA((2,)),
            pltpu.SemaphoreType.DMA((2,)),
            pltpu.SemaphoreType.DMA((2,)),
            pltpu.VMEM((n_rs, chunk, d_model), jnp.bfloat16),
            pltpu.SemaphoreType.DMA((n_rs,)),
            pltpu.SemaphoreType.DMA((n_rs,)),
            pltpu.SemaphoreType.DMA((n_ag,)),
            pltpu.SemaphoreType.DMA((n_ag,)),
            pltpu.SemaphoreType.DMA((n_ag,)),
            pltpu.SemaphoreType.DMA((n_ag,)),
        ],
        compiler_params=pltpu.CompilerParams(
            collective_id=0,
            vmem_limit_bytes=63 * 1024 * 1024,
        ),
    )(x, Wg, Wu, Wd)


# device time: 110277 ns/iter; 1.7399x vs baseline; 1.0166x over previous
import jax
import jax.numpy as jnp
from jax import lax
from jax.experimental import pallas as pl
from jax.experimental.pallas import tpu as pltpu

N_DEV = 4


def kernel(x, Wg, Wu, Wd):
    m, d_model = x.shape
    hid = Wg.shape[1]
    wp = hid // 4
    chunk = m // N_DEV
    dp = chunk
    half = d_model // 2
    n_rs = N_DEV - 1
    n_ag = N_DEV - 1

    def body(x_hbm, wg_hbm, wu_hbm, wd_hbm, out_ref,
             wg_bf, wu_bf, wd_bf, stage_w, stage_d, stage_x,
             wsem, dsem, xsem,
             recv_buf, send_sems, recv_sems,
             ag_ssem_r, ag_rsem_r, ag_ssem_l, ag_rsem_l):
        my = lax.axis_index("i")
        left = lax.rem(my + N_DEV - 1, N_DEV)
        right = lax.rem(my + 1, N_DEV)

        barrier_sem = pltpu.get_barrier_semaphore()
        for nbr in (left, right):
            pl.semaphore_signal(
                barrier_sem, inc=1,
                device_id=(nbr,), device_id_type=pl.DeviceIdType.MESH,
            )
        pl.semaphore_wait(barrier_sem, 2)

        def w_dma(idx):
            ref = wg_hbm if idx < 4 else wu_hbm
            j = idx % 4
            return pltpu.make_async_copy(
                ref.at[:, j * wp:(j + 1) * wp], stage_w.at[idx % 2],
                wsem.at[idx % 2])

        def w_convert(idx):
            dst = wg_bf if idx < 4 else wu_bf
            j = idx % 4
            dst[:, j * wp:(j + 1) * wp] = stage_w[idx % 2].astype(jnp.bfloat16)

        def d_dma(idx):
            return pltpu.make_async_copy(
                wd_hbm.at[idx * dp:(idx + 1) * dp, :], stage_d.at[idx % 2],
                dsem.at[idx % 2])

        def d_convert(idx):
            wd_bf[idx * dp:(idx + 1) * dp, :] = (
                stage_d[idx % 2].astype(jnp.bfloat16))

        def x_dma(c, k):
            return pltpu.make_async_copy(
                x_hbm.at[pl.ds(c * chunk, chunk), :], stage_x.at[k % 2],
                xsem.at[k % 2])

        chunk_order = [lax.rem(my - k + N_DEV, N_DEV) for k in range(N_DEV)]

        x_dma(chunk_order[0], 0).start()
        x_dma(chunk_order[1], 1).start()
        w_dma(0).start()
        w_dma(1).start()
        d_dma(0).start()
        d_dma(1).start()
        for idx in range(4):
            w_dma(idx).wait()
            w_convert(idx)
            if idx + 2 < 8:
                w_dma(idx + 2).start()

        def compute_chunk(k):
            c = chunk_order[k]
            x_dma(c, k).wait()
            xc = stage_x[k % 2].astype(jnp.bfloat16)
            if k + 2 < N_DEV:
                x_dma(chunk_order[k + 2], k + 2).start()
            gate = jnp.dot(xc, wg_bf[:, :], preferred_element_type=jnp.float32)
            if k == 0:
                for idx in range(4, 8):
                    w_dma(idx).wait()
                    w_convert(idx)
                    if idx + 2 < 8:
                        w_dma(idx + 2).start()
            up = jnp.dot(xc, wu_bf[:, :], preferred_element_type=jnp.float32)
            h = (gate * (up * jax.nn.sigmoid(up))).astype(jnp.bfloat16)
            if k == 0:
                for idx in range(8):
                    d_dma(idx).wait()
                    d_convert(idx)
                    if idx + 2 < 8:
                        d_dma(idx + 2).start()
            out_ref[pl.ds(c * chunk, chunk), :] = jnp.dot(
                h, wd_bf[:, :], preferred_element_type=jnp.float32
            ).astype(jnp.bfloat16)

        compute_chunk(0)
        for s in range(n_rs):
            c_send = chunk_order[s]
            c_recv = chunk_order[s + 1]
            rdma = pltpu.make_async_remote_copy(
                src_ref=out_ref.at[pl.ds(c_send * chunk, chunk), :],
                dst_ref=recv_buf.at[s],
                send_sem=send_sems.at[s],
                recv_sem=recv_sems.at[s],
                device_id=(right,),
                device_id_type=pl.DeviceIdType.MESH,
            )
            rdma.start()
            compute_chunk(s + 1)
            rdma.wait()
            dst = pl.ds(c_recv * chunk, chunk)
            out_ref[dst, :] = out_ref[dst, :] + recv_buf[s, :, :]

        for s in range(n_ag):
            c_r = lax.rem(my + 1 - s + N_DEV, N_DEV)
            c_l = lax.rem(my + 1 + s, N_DEV)
            rows_r = pl.ds(c_r * chunk, chunk)
            rows_l = pl.ds(c_l * chunk, chunk)
            rdma_r = pltpu.make_async_remote_copy(
                src_ref=out_ref.at[rows_r, 0:half],
                dst_ref=out_ref.at[rows_r, 0:half],
                send_sem=ag_ssem_r.at[s],
                recv_sem=ag_rsem_r.at[s],
                device_id=(right,),
                device_id_type=pl.DeviceIdType.MESH,
            )
            rdma_l = pltpu.make_async_remote_copy(
                src_ref=out_ref.at[rows_l, half:],
                dst_ref=out_ref.at[rows_l, half:],
                send_sem=ag_ssem_l.at[s],
                recv_sem=ag_rsem_l.at[s],
                device_id=(left,),
                device_id_type=pl.DeviceIdType.MESH,
            )
            rdma_r.start()
            rdma_l.start()
            rdma_r.wait()
            rdma_l.wait()

    return pl.pallas_call(
        body,
        out_shape=jax.ShapeDtypeStruct((m, d_model), jnp.bfloat16),
        in_specs=[pl.BlockSpec(memory_space=pl.ANY)] * 4,
        out_specs=pl.BlockSpec(memory_space=pltpu.VMEM),
        scratch_shapes=[
            pltpu.VMEM((m, hid), jnp.bfloat16),
            pltpu.VMEM((m, hid), jnp.bfloat16),
            pltpu.VMEM((hid, d_model), jnp.bfloat16),
            pltpu.VMEM((2, m, wp), jnp.float32),
            pltpu.VMEM((2, dp, d_model), jnp.float32),
            pltpu.VMEM((2, chunk, d_model), jnp.float32),
            pltpu.SemaphoreType.DMA((2,)),
            pltpu.SemaphoreType.DMA((2,)),
            pltpu.SemaphoreType.DMA((2,)),
            pltpu.VMEM((n_rs, chunk, d_model), jnp.bfloat16),
            pltpu.SemaphoreType.DMA((n_rs,)),
            pltpu.SemaphoreType.DMA((n_rs,)),
            pltpu.SemaphoreType.DMA((n_ag,)),
            pltpu.SemaphoreType.DMA((n_ag,)),
            pltpu.SemaphoreType.DMA((n_ag,)),
            pltpu.SemaphoreType.DMA((n_ag,)),
        ],
        compiler_params=pltpu.CompilerParams(
            collective_id=0,
            vmem_limit_bytes=63 * 1024 * 1024,
        ),
    )(x, Wg, Wu, Wd)
